# Initial kernel scaffold; baseline (speedup 1.0000x reference)
import jax
import jax.numpy as jnp
from jax.experimental import pallas as pl


def kernel(entity_emb, gc1_w, gc1_b, gc2_w, gc2_b, rel_w, rel_b, att_w, att_b, img_w, img_b, name_w, name_b, char_w, char_b, ocr_w, ocr_b, fusion_w, input_idx, adj, img_features, rel_features, att_features, name_features, char_features, ocr_features):
    raise NotImplementedError("write your pallas kernel here")


# speedup vs baseline: 1.1066x; 1.1066x over previous
import functools

import jax
import jax.numpy as jnp
from jax.experimental import pallas as pl
from jax.experimental.pallas import tpu as pltpu

_VMEM_LIMIT = 48 * 1024 * 1024
_ROW_PAD = 256


def _round_up(x, m):
    return ((x + m - 1) // m) * m


def _row_block(n_pad):
    if n_pad >= 1024 and n_pad % 512 == 0:
        return 512
    return min(256, n_pad)


def _k_block(n_pad):
    for c in (1024, 512, 256):
        if c <= n_pad and n_pad % c == 0:
            return c
    return n_pad


def _pad_rows(x, n_pad):
    n = x.shape[0]
    if n == n_pad:
        return x
    return jnp.pad(x, ((0, n_pad - n),) + ((0, 0),) * (x.ndim - 1))


def _layer1_kernel(adj_ref, entx_ref, w1_ref, b1_ref, w2_ref, o_ref, acc_ref):
    k = pl.program_id(1)

    @pl.when(k == 0)
    def _():
        acc_ref[...] = jnp.zeros_like(acc_ref)

    xw1 = jnp.dot(entx_ref[...], w1_ref[...],
                  preferred_element_type=jnp.float32)
    acc_ref[...] += jnp.dot(adj_ref[...].astype(jnp.bfloat16),
                            xw1.astype(jnp.bfloat16),
                            preferred_element_type=jnp.float32)

    @pl.when(k == pl.num_programs(1) - 1)
    def _():
        h = jnp.maximum(acc_ref[...] + b1_ref[...], 0.0)
        o_ref[...] = jnp.dot(h, w2_ref[...],
                             preferred_element_type=jnp.float32
                             ).astype(jnp.bfloat16)


def _layer2_modal_kernel(bk, wn_ref, adj_ref, hw2_ref, b2_ref,
                         img_x, rel_x, att_x, name_x, char_x, ocr_x,
                         img_w, img_b, rel_w, rel_b, att_w, att_b,
                         name_w, name_b, char_w, char_b, ocr_w, ocr_b,
                         gph_o, img_o, rel_o, att_o, name_o, char_o,
                         joint_o, ocr_o, acc_ref):
    k = pl.program_id(1)

    @pl.when(k == 0)
    def _():
        acc_ref[...] = jnp.zeros_like(acc_ref)

    start = pl.multiple_of(k * bk, bk)
    acc_ref[...] += jnp.dot(adj_ref[...].astype(jnp.bfloat16),
                            hw2_ref[pl.ds(start, bk), :],
                            preferred_element_type=jnp.float32)

    @pl.when(k == pl.num_programs(1) - 1)
    def _():
        gph = acc_ref[...] + b2_ref[...]
        gph_o[...] = gph

        def proj(x_ref, w_ref, b_ref):
            return jnp.dot(x_ref[...], w_ref[...],
                           preferred_element_type=jnp.float32) + b_ref[...]

        def l2n(e):
            ss = jnp.sum(e * e, axis=1, keepdims=True)
            return e * jax.lax.rsqrt(jnp.maximum(ss, 1e-24))

        img_e = proj(img_x, img_w, img_b)
        rel_e = proj(rel_x, rel_w, rel_b)
        att_e = proj(att_x, att_w, att_b)
        name_e = proj(name_x, name_w, name_b)
        char_e = proj(char_x, char_w, char_b)
        ocr_e = proj(ocr_x, ocr_w, ocr_b)

        img_o[...] = img_e
        rel_o[...] = rel_e
        att_o[...] = att_e
        name_o[...] = name_e
        char_o[...] = char_e
        ocr_o[...] = ocr_e

        off = 0
        for e, wn_idx in ((img_e, 0), (att_e, 1), (rel_e, 2), (gph, 3),
                          (name_e, 4), (char_e, 5), (ocr_e, 6)):
            d = e.shape[1]
            joint_o[:, off:off + d] = l2n(e) * wn_ref[wn_idx]
            off += d


def kernel(entity_emb, gc1_w, gc1_b, gc2_w, gc2_b, rel_w, rel_b, att_w, att_b,
           img_w, img_b, name_w, name_b, char_w, char_b, ocr_w, ocr_b,
           fusion_w, input_idx, adj, img_features, rel_features, att_features,
           name_features, char_features, ocr_features):
    n = adj.shape[0]
    n_pad = _round_up(max(n, _ROW_PAD), _ROW_PAD)
    bm = _row_block(n_pad)
    bk = _k_block(n_pad)
    grid = (n_pad // bm, n_pad // bk)

    ent_x = _pad_rows(entity_emb[input_idx], n_pad)
    adj_p = _pad_rows(
        jnp.pad(adj, ((0, 0), (0, n_pad - n))) if n != n_pad else adj, n_pad)
    img_x = _pad_rows(img_features, n_pad)
    rel_x = _pad_rows(rel_features, n_pad)
    att_x = _pad_rows(att_features, n_pad)
    name_x = _pad_rows(name_features, n_pad)
    char_x = _pad_rows(char_features, n_pad)
    ocr_x = _pad_rows(ocr_features, n_pad)

    d_in = ent_x.shape[1]
    nhid = gc1_w.shape[1]
    nout = gc2_w.shape[1]
    b1 = gc1_b.reshape(1, -1)
    b2 = gc2_b.reshape(1, -1)

    cost1 = pl.CostEstimate(
        flops=2 * n_pad * n_pad * nhid, transcendentals=0,
        bytes_accessed=4 * n_pad * n_pad + 2 * n_pad * nout)
    hw2 = pl.pallas_call(
        _layer1_kernel,
        grid=grid,
        in_specs=[
            pl.BlockSpec((bm, bk), lambda i, k: (i, k)),
            pl.BlockSpec((bk, d_in), lambda i, k: (k, 0)),
            pl.BlockSpec((d_in, nhid), lambda i, k: (0, 0)),
            pl.BlockSpec((1, nhid), lambda i, k: (0, 0)),
            pl.BlockSpec((nhid, nout), lambda i, k: (0, 0)),
        ],
        out_specs=pl.BlockSpec((bm, nout), lambda i, k: (i, 0)),
        out_shape=jax.ShapeDtypeStruct((n_pad, nout), jnp.bfloat16),
        scratch_shapes=[pltpu.VMEM((bm, nhid), jnp.float32)],
        compiler_params=pltpu.CompilerParams(
            dimension_semantics=("parallel", "arbitrary"),
            vmem_limit_bytes=_VMEM_LIMIT),
        cost_estimate=cost1,
    )(adj_p, ent_x, gc1_w, b1, gc2_w)

    weight_norm = jax.nn.softmax(fusion_w, axis=0)[:, 0]

    d_img = img_w.shape[1]
    d_rel = rel_w.shape[1]
    d_att = att_w.shape[1]
    d_name = name_w.shape[1]
    d_char = char_w.shape[1]
    d_ocr = ocr_w.shape[1]
    d_joint = d_img + d_att + d_rel + nout + d_name + d_char + d_ocr

    def row_spec(d):
        return pl.BlockSpec((bm, d), lambda i, k: (i, 0))

    def pinned(shape):
        return pl.BlockSpec(shape, lambda i, k: (0, 0))

    in_specs = [
        pl.BlockSpec(memory_space=pltpu.MemorySpace.SMEM),
        pl.BlockSpec((bm, bk), lambda i, k: (i, k)),
        pinned((n_pad, nout)),
        pinned((1, nout)),
        row_spec(img_x.shape[1]), row_spec(rel_x.shape[1]),
        row_spec(att_x.shape[1]), row_spec(name_x.shape[1]),
        row_spec(char_x.shape[1]), row_spec(ocr_x.shape[1]),
        pinned(img_w.shape), pinned((1, d_img)),
        pinned(rel_w.shape), pinned((1, d_rel)),
        pinned(att_w.shape), pinned((1, d_att)),
        pinned(name_w.shape), pinned((1, d_name)),
        pinned(char_w.shape), pinned((1, d_char)),
        pinned(ocr_w.shape), pinned((1, d_ocr)),
    ]
    out_specs = (row_spec(nout), row_spec(d_img), row_spec(d_rel),
                 row_spec(d_att), row_spec(d_name), row_spec(d_char),
                 row_spec(d_joint), row_spec(d_ocr))
    out_shape = tuple(jax.ShapeDtypeStruct((n_pad, d), jnp.float32)
                      for d in (nout, d_img, d_rel, d_att, d_name, d_char,
                                d_joint, d_ocr))

    flops2 = (2 * n_pad * n_pad * nout
              + 2 * n_pad * (img_x.shape[1] * d_img + rel_x.shape[1] * d_rel
                             + att_x.shape[1] * d_att
                             + name_x.shape[1] * d_name
                             + char_x.shape[1] * d_char
                             + ocr_x.shape[1] * d_ocr))
    bytes2 = (4 * n_pad * n_pad
              + 4 * n_pad * (img_x.shape[1] + rel_x.shape[1] + att_x.shape[1]
                             + name_x.shape[1] + char_x.shape[1]
                             + ocr_x.shape[1])
              + 4 * n_pad * (nout + d_img + d_rel + d_att + d_name + d_char
                             + d_joint + d_ocr))
    cost2 = pl.CostEstimate(flops=flops2, transcendentals=7 * n_pad,
                            bytes_accessed=bytes2)

    outs = pl.pallas_call(
        functools.partial(_layer2_modal_kernel, bk),
        grid=grid,
        in_specs=in_specs,
        out_specs=out_specs,
        out_shape=out_shape,
        scratch_shapes=[pltpu.VMEM((bm, nout), jnp.float32)],
        compiler_params=pltpu.CompilerParams(
            dimension_semantics=("parallel", "arbitrary"),
            vmem_limit_bytes=_VMEM_LIMIT),
        cost_estimate=cost2,
    )(weight_norm, adj_p, hw2, b2,
      img_x, rel_x, att_x, name_x, char_x, ocr_x,
      img_w, img_b.reshape(1, -1), rel_w, rel_b.reshape(1, -1),
      att_w, att_b.reshape(1, -1), name_w, name_b.reshape(1, -1),
      char_w, char_b.reshape(1, -1), ocr_w, ocr_b.reshape(1, -1))

    gph_o, img_o, rel_o, att_o, name_o, char_o, joint_o, ocr_o = outs
    return (gph_o[:n], img_o[:n], rel_o[:n], att_o[:n], name_o[:n],
            char_o[:n], joint_o[:n], ocr_o[:n])


# speedup vs baseline: 1.1402x; 1.0304x over previous
import functools

import jax
import jax.numpy as jnp
from jax.experimental import pallas as pl
from jax.experimental.pallas import tpu as pltpu

_VMEM_LIMIT = 60 * 1024 * 1024
_ROW_PAD = 256


def _round_up(x, m):
    return ((x + m - 1) // m) * m


def _row_block(n_pad):
    if n_pad >= 1024 and n_pad % 512 == 0:
        return 512
    return min(256, n_pad)


def _k_block(n_pad):
    for c in (1024, 512, 256):
        if c <= n_pad and n_pad % c == 0:
            return c
    return n_pad


def _pad_rows(x, n_pad):
    n = x.shape[0]
    if n == n_pad:
        return x
    return jnp.pad(x, ((0, n_pad - n),) + ((0, 0),) * (x.ndim - 1))


def _fused_kernel(bm, bk, wn_ref, adj_ref, entx_ref, w1_ref, b1_ref, w2_ref,
                  b2_ref,
                  img_x, rel_x, att_x, name_x, char_x, ocr_x,
                  img_w, img_b, rel_w, rel_b, att_w, att_b,
                  name_w, name_b, char_w, char_b, ocr_w, ocr_b,
                  gph_o, img_o, rel_o, att_o, name_o, char_o,
                  joint_o, ocr_o,
                  adj_c_ref, hw2_ref, acc_ref):
    l = pl.program_id(0)
    i = pl.program_id(1)
    k = pl.program_id(2)
    nk = pl.num_programs(2)
    row0 = pl.multiple_of(i * bm, bm)
    col0 = pl.multiple_of(k * bk, bk)

    @pl.when(k == 0)
    def _():
        acc_ref[...] = jnp.zeros_like(acc_ref)

    @pl.when(l == 0)
    def _():
        a16 = adj_ref[...].astype(jnp.bfloat16)
        adj_c_ref[pl.ds(row0, bm), pl.ds(col0, bk)] = a16
        xw1 = jnp.dot(entx_ref[...], w1_ref[...],
                      preferred_element_type=jnp.float32)
        acc_ref[...] += jnp.dot(a16, xw1.astype(jnp.bfloat16),
                                preferred_element_type=jnp.float32)

        @pl.when(k == nk - 1)
        def _():
            h = jnp.maximum(acc_ref[...] + b1_ref[...], 0.0)
            hw2_ref[pl.ds(row0, bm), :] = jnp.dot(
                h, w2_ref[...], preferred_element_type=jnp.float32
            ).astype(jnp.bfloat16)

    @pl.when(l == 1)
    def _():
        a16 = adj_c_ref[pl.ds(row0, bm), pl.ds(col0, bk)]
        acc_ref[...] += jnp.dot(a16, hw2_ref[pl.ds(col0, bk), :],
                                preferred_element_type=jnp.float32)

        @pl.when(k == nk - 1)
        def _():
            gph = acc_ref[...] + b2_ref[...]
            gph_o[...] = gph

            def proj(x_ref, w_ref, b_ref):
                return jnp.dot(x_ref[...], w_ref[...],
                               preferred_element_type=jnp.float32) + b_ref[...]

            def l2n(e):
                ss = jnp.sum(e * e, axis=1, keepdims=True)
                return e * jax.lax.rsqrt(jnp.maximum(ss, 1e-24))

            img_e = proj(img_x, img_w, img_b)
            rel_e = proj(rel_x, rel_w, rel_b)
            att_e = proj(att_x, att_w, att_b)
            name_e = proj(name_x, name_w, name_b)
            char_e = proj(char_x, char_w, char_b)
            ocr_e = proj(ocr_x, ocr_w, ocr_b)

            img_o[...] = img_e
            rel_o[...] = rel_e
            att_o[...] = att_e
            name_o[...] = name_e
            char_o[...] = char_e
            ocr_o[...] = ocr_e

            off = 0
            for e, wn_idx in ((img_e, 0), (att_e, 1), (rel_e, 2), (gph, 3),
                              (name_e, 4), (char_e, 5), (ocr_e, 6)):
                d = e.shape[1]
                joint_o[:, off:off + d] = l2n(e) * wn_ref[wn_idx]
                off += d


def kernel(entity_emb, gc1_w, gc1_b, gc2_w, gc2_b, rel_w, rel_b, att_w, att_b,
           img_w, img_b, name_w, name_b, char_w, char_b, ocr_w, ocr_b,
           fusion_w, input_idx, adj, img_features, rel_features, att_features,
           name_features, char_features, ocr_features):
    n = adj.shape[0]
    n_pad = _round_up(max(n, _ROW_PAD), _ROW_PAD)
    bm = _row_block(n_pad)
    bk = _k_block(n_pad)
    grid = (2, n_pad // bm, n_pad // bk)

    ent_x = _pad_rows(entity_emb[input_idx], n_pad)
    adj_p = _pad_rows(
        jnp.pad(adj, ((0, 0), (0, n_pad - n))) if n != n_pad else adj, n_pad)
    img_x = _pad_rows(img_features, n_pad)
    rel_x = _pad_rows(rel_features, n_pad)
    att_x = _pad_rows(att_features, n_pad)
    name_x = _pad_rows(name_features, n_pad)
    char_x = _pad_rows(char_features, n_pad)
    ocr_x = _pad_rows(ocr_features, n_pad)

    d_in = ent_x.shape[1]
    nhid = gc1_w.shape[1]
    nout = gc2_w.shape[1]
    b1 = gc1_b.reshape(1, -1)
    b2 = gc2_b.reshape(1, -1)

    weight_norm = jax.nn.softmax(fusion_w, axis=0)[:, 0]

    d_img = img_w.shape[1]
    d_rel = rel_w.shape[1]
    d_att = att_w.shape[1]
    d_name = name_w.shape[1]
    d_char = char_w.shape[1]
    d_ocr = ocr_w.shape[1]
    d_joint = d_img + d_att + d_rel + nout + d_name + d_char + d_ocr

    def row_spec(d):
        return pl.BlockSpec((bm, d), lambda l, i, k: (i * l, 0))

    def pinned(shape):
        return pl.BlockSpec(shape, lambda l, i, k: (0, 0))

    in_specs = [
        pl.BlockSpec(memory_space=pltpu.MemorySpace.SMEM),
        pl.BlockSpec((bm, bk), lambda l, i, k: (i * (1 - l), k * (1 - l))),
        pl.BlockSpec((bk, d_in), lambda l, i, k: (k * (1 - l), 0)),
        pinned((d_in, nhid)),
        pinned((1, nhid)),
        pinned((nhid, nout)),
        pinned((1, nout)),
        row_spec(img_x.shape[1]), row_spec(rel_x.shape[1]),
        row_spec(att_x.shape[1]), row_spec(name_x.shape[1]),
        row_spec(char_x.shape[1]), row_spec(ocr_x.shape[1]),
        pinned(img_w.shape), pinned((1, d_img)),
        pinned(rel_w.shape), pinned((1, d_rel)),
        pinned(att_w.shape), pinned((1, d_att)),
        pinned(name_w.shape), pinned((1, d_name)),
        pinned(char_w.shape), pinned((1, d_char)),
        pinned(ocr_w.shape), pinned((1, d_ocr)),
    ]
    out_specs = (row_spec(nout), row_spec(d_img), row_spec(d_rel),
                 row_spec(d_att), row_spec(d_name), row_spec(d_char),
                 row_spec(d_joint), row_spec(d_ocr))
    out_shape = tuple(jax.ShapeDtypeStruct((n_pad, d), jnp.float32)
                      for d in (nout, d_img, d_rel, d_att, d_name, d_char,
                                d_joint, d_ocr))

    flops = (2 * n_pad * n_pad * (nhid + nout)
             + 2 * n_pad * (img_x.shape[1] * d_img + rel_x.shape[1] * d_rel
                            + att_x.shape[1] * d_att
                            + name_x.shape[1] * d_name
                            + char_x.shape[1] * d_char
                            + ocr_x.shape[1] * d_ocr))
    bytes_acc = (4 * n_pad * n_pad
                 + 4 * n_pad * (img_x.shape[1] + rel_x.shape[1]
                                + att_x.shape[1] + name_x.shape[1]
                                + char_x.shape[1] + ocr_x.shape[1])
                 + 4 * n_pad * (nout + d_img + d_rel + d_att + d_name + d_char
                                + d_joint + d_ocr))
    cost = pl.CostEstimate(flops=flops, transcendentals=7 * n_pad,
                           bytes_accessed=bytes_acc)

    outs = pl.pallas_call(
        functools.partial(_fused_kernel, bm, bk),
        grid=grid,
        in_specs=in_specs,
        out_specs=out_specs,
        out_shape=out_shape,
        scratch_shapes=[pltpu.VMEM((n_pad, n_pad), jnp.bfloat16),
                        pltpu.VMEM((n_pad, nout), jnp.bfloat16),
                        pltpu.VMEM((bm, nout), jnp.float32)],
        compiler_params=pltpu.CompilerParams(
            dimension_semantics=("arbitrary", "arbitrary", "arbitrary"),
            vmem_limit_bytes=_VMEM_LIMIT),
        cost_estimate=cost,
    )(weight_norm, adj_p, ent_x, gc1_w, b1, gc2_w, b2,
      img_x, rel_x, att_x, name_x, char_x, ocr_x,
      img_w, img_b.reshape(1, -1), rel_w, rel_b.reshape(1, -1),
      att_w, att_b.reshape(1, -1), name_w, name_b.reshape(1, -1),
      char_w, char_b.reshape(1, -1), ocr_w, ocr_b.reshape(1, -1))

    gph_o, img_o, rel_o, att_o, name_o, char_o, joint_o, ocr_o = outs
    return (gph_o[:n], img_o[:n], rel_o[:n], att_o[:n], name_o[:n],
            char_o[:n], joint_o[:n], ocr_o[:n])


# speedup vs baseline: 1.2631x; 1.1078x over previous
import functools

import jax
import jax.numpy as jnp
from jax.experimental import pallas as pl
from jax.experimental.pallas import tpu as pltpu

_VMEM_LIMIT = 60 * 1024 * 1024
_ROW_PAD = 256


def _round_up(x, m):
    return ((x + m - 1) // m) * m


def _pad_rows(x, n_pad):
    n = x.shape[0]
    if n == n_pad:
        return x
    return jnp.pad(x, ((0, n_pad - n),) + ((0, 0),) * (x.ndim - 1))


def _layer1_kernel(adj_ref, entx_ref, w1_ref, b1_ref, w2_ref, o_ref):
    xw1 = jnp.dot(entx_ref[...], w1_ref[...],
                  preferred_element_type=jnp.float32)
    acc = jnp.dot(adj_ref[...].astype(jnp.bfloat16),
                  xw1.astype(jnp.bfloat16),
                  preferred_element_type=jnp.float32)
    h = jnp.maximum(acc + b1_ref[...], 0.0)
    o_ref[...] = jnp.dot(h, w2_ref[...],
                         preferred_element_type=jnp.float32
                         ).astype(jnp.bfloat16)


def _layer2_modal_kernel(wn_ref, adj_ref, hw2_ref, b2_ref,
                         img_x, rel_x, att_x, name_x, char_x, ocr_x,
                         img_w, img_b, rel_w, rel_b, att_w, att_b,
                         name_w, name_b, char_w, char_b, ocr_w, ocr_b,
                         gph_o, img_o, rel_o, att_o, name_o, char_o,
                         joint_o, ocr_o):
    gph = jnp.dot(adj_ref[...].astype(jnp.bfloat16), hw2_ref[...],
                  preferred_element_type=jnp.float32) + b2_ref[...]
    gph_o[...] = gph

    def proj(x_ref, w_ref, b_ref):
        return jnp.dot(x_ref[...], w_ref[...],
                       preferred_element_type=jnp.float32) + b_ref[...]

    def l2n(e):
        ss = jnp.sum(e * e, axis=1, keepdims=True)
        return e * jax.lax.rsqrt(jnp.maximum(ss, 1e-24))

    img_e = proj(img_x, img_w, img_b)
    rel_e = proj(rel_x, rel_w, rel_b)
    att_e = proj(att_x, att_w, att_b)
    name_e = proj(name_x, name_w, name_b)
    char_e = proj(char_x, char_w, char_b)
    ocr_e = proj(ocr_x, ocr_w, ocr_b)

    img_o[...] = img_e
    rel_o[...] = rel_e
    att_o[...] = att_e
    name_o[...] = name_e
    char_o[...] = char_e
    ocr_o[...] = ocr_e

    off = 0
    for e, wn_idx in ((img_e, 0), (att_e, 1), (rel_e, 2), (gph, 3),
                      (name_e, 4), (char_e, 5), (ocr_e, 6)):
        d = e.shape[1]
        joint_o[:, off:off + d] = l2n(e) * wn_ref[wn_idx]
        off += d


def kernel(entity_emb, gc1_w, gc1_b, gc2_w, gc2_b, rel_w, rel_b, att_w, att_b,
           img_w, img_b, name_w, name_b, char_w, char_b, ocr_w, ocr_b,
           fusion_w, input_idx, adj, img_features, rel_features, att_features,
           name_features, char_features, ocr_features):
    n = adj.shape[0]
    n_pad = _round_up(max(n, _ROW_PAD), _ROW_PAD)
    bm1 = 1024 if n_pad % 1024 == 0 else (512 if n_pad % 512 == 0 else 256)
    bm2 = 512 if n_pad % 512 == 0 else 256

    ent_x = _pad_rows(entity_emb[input_idx], n_pad)
    adj_p = _pad_rows(
        jnp.pad(adj, ((0, 0), (0, n_pad - n))) if n != n_pad else adj, n_pad)
    img_x = _pad_rows(img_features, n_pad)
    rel_x = _pad_rows(rel_features, n_pad)
    att_x = _pad_rows(att_features, n_pad)
    name_x = _pad_rows(name_features, n_pad)
    char_x = _pad_rows(char_features, n_pad)
    ocr_x = _pad_rows(ocr_features, n_pad)

    d_in = ent_x.shape[1]
    nhid = gc1_w.shape[1]
    nout = gc2_w.shape[1]
    b1 = gc1_b.reshape(1, -1)
    b2 = gc2_b.reshape(1, -1)

    cost1 = pl.CostEstimate(
        flops=2 * n_pad * n_pad * nhid, transcendentals=0,
        bytes_accessed=4 * n_pad * n_pad + 2 * n_pad * nout)
    hw2 = pl.pallas_call(
        _layer1_kernel,
        grid=(n_pad // bm1,),
        in_specs=[
            pl.BlockSpec((bm1, n_pad), lambda i: (i, 0)),
            pl.BlockSpec((n_pad, d_in), lambda i: (0, 0)),
            pl.BlockSpec((d_in, nhid), lambda i: (0, 0)),
            pl.BlockSpec((1, nhid), lambda i: (0, 0)),
            pl.BlockSpec((nhid, nout), lambda i: (0, 0)),
        ],
        out_specs=pl.BlockSpec((bm1, nout), lambda i: (i, 0)),
        out_shape=jax.ShapeDtypeStruct((n_pad, nout), jnp.bfloat16),
        compiler_params=pltpu.CompilerParams(
            dimension_semantics=("parallel",),
            vmem_limit_bytes=_VMEM_LIMIT),
        cost_estimate=cost1,
    )(adj_p, ent_x, gc1_w, b1, gc2_w)

    weight_norm = jax.nn.softmax(fusion_w, axis=0)[:, 0]

    d_img = img_w.shape[1]
    d_rel = rel_w.shape[1]
    d_att = att_w.shape[1]
    d_name = name_w.shape[1]
    d_char = char_w.shape[1]
    d_ocr = ocr_w.shape[1]
    d_joint = d_img + d_att + d_rel + nout + d_name + d_char + d_ocr

    def row_spec(d):
        return pl.BlockSpec((bm2, d), lambda i: (i, 0))

    def pinned(shape):
        return pl.BlockSpec(shape, lambda i: (0, 0))

    in_specs = [
        pl.BlockSpec(memory_space=pltpu.MemorySpace.SMEM),
        pl.BlockSpec((bm2, n_pad), lambda i: (i, 0)),
        pinned((n_pad, nout)),
        pinned((1, nout)),
        row_spec(img_x.shape[1]), row_spec(rel_x.shape[1]),
        row_spec(att_x.shape[1]), row_spec(name_x.shape[1]),
        row_spec(char_x.shape[1]), row_spec(ocr_x.shape[1]),
        pinned(img_w.shape), pinned((1, d_img)),
        pinned(rel_w.shape), pinned((1, d_rel)),
        pinned(att_w.shape), pinned((1, d_att)),
        pinned(name_w.shape), pinned((1, d_name)),
        pinned(char_w.shape), pinned((1, d_char)),
        pinned(ocr_w.shape), pinned((1, d_ocr)),
    ]
    out_specs = (row_spec(nout), row_spec(d_img), row_spec(d_rel),
                 row_spec(d_att), row_spec(d_name), row_spec(d_char),
                 row_spec(d_joint), row_spec(d_ocr))
    out_shape = tuple(jax.ShapeDtypeStruct((n_pad, d), jnp.float32)
                      for d in (nout, d_img, d_rel, d_att, d_name, d_char,
                                d_joint, d_ocr))

    flops2 = (2 * n_pad * n_pad * nout
              + 2 * n_pad * (img_x.shape[1] * d_img + rel_x.shape[1] * d_rel
                             + att_x.shape[1] * d_att
                             + name_x.shape[1] * d_name
                             + char_x.shape[1] * d_char
                             + ocr_x.shape[1] * d_ocr))
    bytes2 = (4 * n_pad * n_pad
              + 4 * n_pad * (img_x.shape[1] + rel_x.shape[1] + att_x.shape[1]
                             + name_x.shape[1] + char_x.shape[1]
                             + ocr_x.shape[1])
              + 4 * n_pad * (nout + d_img + d_rel + d_att + d_name + d_char
                             + d_joint + d_ocr))
    cost2 = pl.CostEstimate(flops=flops2, transcendentals=7 * n_pad,
                            bytes_accessed=bytes2)

    outs = pl.pallas_call(
        _layer2_modal_kernel,
        grid=(n_pad // bm2,),
        in_specs=in_specs,
        out_specs=out_specs,
        out_shape=out_shape,
        compiler_params=pltpu.CompilerParams(
            dimension_semantics=("parallel",),
            vmem_limit_bytes=_VMEM_LIMIT),
        cost_estimate=cost2,
    )(weight_norm, adj_p, hw2, b2,
      img_x, rel_x, att_x, name_x, char_x, ocr_x,
      img_w, img_b.reshape(1, -1), rel_w, rel_b.reshape(1, -1),
      att_w, att_b.reshape(1, -1), name_w, name_b.reshape(1, -1),
      char_w, char_b.reshape(1, -1), ocr_w, ocr_b.reshape(1, -1))

    gph_o, img_o, rel_o, att_o, name_o, char_o, joint_o, ocr_o = outs
    return (gph_o[:n], img_o[:n], rel_o[:n], att_o[:n], name_o[:n],
            char_o[:n], joint_o[:n], ocr_o[:n])


# speedup vs baseline: 1.2888x; 1.0204x over previous
import functools

import jax
import jax.numpy as jnp
from jax.experimental import pallas as pl
from jax.experimental.pallas import tpu as pltpu

_VMEM_LIMIT = 60 * 1024 * 1024
_ROW_PAD = 256


def _round_up(x, m):
    return ((x + m - 1) // m) * m


def _pad_rows(x, n_pad):
    n = x.shape[0]
    if n == n_pad:
        return x
    return jnp.pad(x, ((0, n_pad - n),) + ((0, 0),) * (x.ndim - 1))


def _fused_kernel(bm, wn_ref, adj_ref, entx_ref, w1_ref, b1_ref, w2_ref,
                  b2_ref,
                  img_x, rel_x, att_x, name_x, char_x, ocr_x,
                  img_w, img_b, rel_w, rel_b, att_w, att_b,
                  name_w, name_b, char_w, char_b, ocr_w, ocr_b,
                  gph_o, img_o, rel_o, att_o, name_o, char_o,
                  joint_o, ocr_o,
                  adj_c_ref, hw2_ref):
    l = pl.program_id(0)
    i = pl.program_id(1)
    row0 = pl.multiple_of(i * bm, bm)

    @pl.when(l == 0)
    def _():
        a16 = adj_ref[...].astype(jnp.bfloat16)
        adj_c_ref[pl.ds(row0, bm), :] = a16
        xw1 = jnp.dot(entx_ref[...], w1_ref[...],
                      preferred_element_type=jnp.float32)
        acc = jnp.dot(a16, xw1.astype(jnp.bfloat16),
                      preferred_element_type=jnp.float32)
        h = jnp.maximum(acc + b1_ref[...], 0.0)
        hw2_ref[pl.ds(row0, bm), :] = jnp.dot(
            h, w2_ref[...], preferred_element_type=jnp.float32
        ).astype(jnp.bfloat16)

    @pl.when(l == 1)
    def _():
        a16 = adj_c_ref[pl.ds(row0, bm), :]
        gph = jnp.dot(a16, hw2_ref[...],
                      preferred_element_type=jnp.float32) + b2_ref[...]
        gph_o[...] = gph

        def proj(x_ref, w_ref, b_ref):
            return jnp.dot(x_ref[...], w_ref[...],
                           preferred_element_type=jnp.float32) + b_ref[...]

        def l2n(e):
            ss = jnp.sum(e * e, axis=1, keepdims=True)
            return e * jax.lax.rsqrt(jnp.maximum(ss, 1e-24))

        img_e = proj(img_x, img_w, img_b)
        rel_e = proj(rel_x, rel_w, rel_b)
        att_e = proj(att_x, att_w, att_b)
        name_e = proj(name_x, name_w, name_b)
        char_e = proj(char_x, char_w, char_b)
        ocr_e = proj(ocr_x, ocr_w, ocr_b)

        img_o[...] = img_e
        rel_o[...] = rel_e
        att_o[...] = att_e
        name_o[...] = name_e
        char_o[...] = char_e
        ocr_o[...] = ocr_e

        off = 0
        for e, wn_idx in ((img_e, 0), (att_e, 1), (rel_e, 2), (gph, 3),
                          (name_e, 4), (char_e, 5), (ocr_e, 6)):
            d = e.shape[1]
            joint_o[:, off:off + d] = l2n(e) * wn_ref[wn_idx]
            off += d


def kernel(entity_emb, gc1_w, gc1_b, gc2_w, gc2_b, rel_w, rel_b, att_w, att_b,
           img_w, img_b, name_w, name_b, char_w, char_b, ocr_w, ocr_b,
           fusion_w, input_idx, adj, img_features, rel_features, att_features,
           name_features, char_features, ocr_features):
    n = adj.shape[0]
    n_pad = _round_up(max(n, _ROW_PAD), _ROW_PAD)
    bm = 256

    ent_x = _pad_rows(entity_emb[input_idx], n_pad)
    adj_p = _pad_rows(
        jnp.pad(adj, ((0, 0), (0, n_pad - n))) if n != n_pad else adj, n_pad)
    img_x = _pad_rows(img_features, n_pad)
    rel_x = _pad_rows(rel_features, n_pad)
    att_x = _pad_rows(att_features, n_pad)
    name_x = _pad_rows(name_features, n_pad)
    char_x = _pad_rows(char_features, n_pad)
    ocr_x = _pad_rows(ocr_features, n_pad)

    d_in = ent_x.shape[1]
    nhid = gc1_w.shape[1]
    nout = gc2_w.shape[1]
    b1 = gc1_b.reshape(1, -1)
    b2 = gc2_b.reshape(1, -1)

    weight_norm = jax.nn.softmax(fusion_w, axis=0)[:, 0]

    d_img = img_w.shape[1]
    d_rel = rel_w.shape[1]
    d_att = att_w.shape[1]
    d_name = name_w.shape[1]
    d_char = char_w.shape[1]
    d_ocr = ocr_w.shape[1]
    d_joint = d_img + d_att + d_rel + nout + d_name + d_char + d_ocr

    def row_spec(d):
        return pl.BlockSpec((bm, d), lambda l, i: (i * l, 0))

    def pinned(shape):
        return pl.BlockSpec(shape, lambda l, i: (0, 0))

    in_specs = [
        pl.BlockSpec(memory_space=pltpu.MemorySpace.SMEM),
        pl.BlockSpec((bm, n_pad), lambda l, i: (i * (1 - l), 0)),
        pinned((n_pad, d_in)),
        pinned((d_in, nhid)),
        pinned((1, nhid)),
        pinned((nhid, nout)),
        pinned((1, nout)),
        row_spec(img_x.shape[1]), row_spec(rel_x.shape[1]),
        row_spec(att_x.shape[1]), row_spec(name_x.shape[1]),
        row_spec(char_x.shape[1]), row_spec(ocr_x.shape[1]),
        pinned(img_w.shape), pinned((1, d_img)),
        pinned(rel_w.shape), pinned((1, d_rel)),
        pinned(att_w.shape), pinned((1, d_att)),
        pinned(name_w.shape), pinned((1, d_name)),
        pinned(char_w.shape), pinned((1, d_char)),
        pinned(ocr_w.shape), pinned((1, d_ocr)),
    ]
    out_specs = (row_spec(nout), row_spec(d_img), row_spec(d_rel),
                 row_spec(d_att), row_spec(d_name), row_spec(d_char),
                 row_spec(d_joint), row_spec(d_ocr))
    out_shape = tuple(jax.ShapeDtypeStruct((n_pad, d), jnp.float32)
                      for d in (nout, d_img, d_rel, d_att, d_name, d_char,
                                d_joint, d_ocr))

    flops = (2 * n_pad * n_pad * (nhid + nout)
             + 2 * n_pad * (img_x.shape[1] * d_img + rel_x.shape[1] * d_rel
                            + att_x.shape[1] * d_att
                            + name_x.shape[1] * d_name
                            + char_x.shape[1] * d_char
                            + ocr_x.shape[1] * d_ocr))
    bytes_acc = (4 * n_pad * n_pad
                 + 4 * n_pad * (img_x.shape[1] + rel_x.shape[1]
                                + att_x.shape[1] + name_x.shape[1]
                                + char_x.shape[1] + ocr_x.shape[1])
                 + 4 * n_pad * (nout + d_img + d_rel + d_att + d_name + d_char
                                + d_joint + d_ocr))
    cost = pl.CostEstimate(flops=flops, transcendentals=7 * n_pad,
                           bytes_accessed=bytes_acc)

    outs = pl.pallas_call(
        functools.partial(_fused_kernel, bm),
        grid=(2, n_pad // bm),
        in_specs=in_specs,
        out_specs=out_specs,
        out_shape=out_shape,
        scratch_shapes=[pltpu.VMEM((n_pad, n_pad), jnp.bfloat16),
                        pltpu.VMEM((n_pad, nout), jnp.bfloat16)],
        compiler_params=pltpu.CompilerParams(
            dimension_semantics=("arbitrary", "arbitrary"),
            vmem_limit_bytes=_VMEM_LIMIT),
        cost_estimate=cost,
    )(weight_norm, adj_p, ent_x, gc1_w, b1, gc2_w, b2,
      img_x, rel_x, att_x, name_x, char_x, ocr_x,
      img_w, img_b.reshape(1, -1), rel_w, rel_b.reshape(1, -1),
      att_w, att_b.reshape(1, -1), name_w, name_b.reshape(1, -1),
      char_w, char_b.reshape(1, -1), ocr_w, ocr_b.reshape(1, -1))

    gph_o, img_o, rel_o, att_o, name_o, char_o, joint_o, ocr_o = outs
    return (gph_o[:n], img_o[:n], rel_o[:n], att_o[:n], name_o[:n],
            char_o[:n], joint_o[:n], ocr_o[:n])
